# Initial kernel scaffold; baseline (speedup 1.0000x reference)
import jax
import jax.numpy as jnp
from jax.experimental import pallas as pl


def kernel(idx, tok_emb, pos_emb):
    raise NotImplementedError("write your pallas kernel here")


# speedup vs baseline: 1.1776x; 1.1776x over previous
import functools

import jax
import jax.numpy as jnp
from jax import lax
from jax.experimental import pallas as pl
from jax.experimental.pallas import tpu as pltpu
from jax.experimental.pallas import tpu_sc as plsc

BATCH = 4
SEQ = 2048
D = 128
TOKENS = BATCH * SEQ
NC, NS = 2, 16
NW = NC * NS
B_PER_W = TOKENS // NW
CHUNK = 128
N_CHUNKS = B_PER_W // CHUNK


def _emb_body(idx_hbm, tok_hbm, pos_hbm, out_hbm, idx_v, rows_v, pos_v,
              gsem0, gsem1, psem):
    c = lax.axis_index("c")
    s = lax.axis_index("s")
    wid = s * NC + c
    base = wid * B_PER_W
    t0 = lax.rem(base, SEQ)

    pos_cp = pltpu.async_copy(pos_hbm.at[pl.ds(t0, B_PER_W)], pos_v, psem)

    pltpu.sync_copy(idx_hbm.at[pl.ds(wid * N_CHUNKS, N_CHUNKS)], idx_v)

    cp0 = pltpu.async_copy(tok_hbm.at[idx_v.at[0]],
                           rows_v.at[pl.ds(0, CHUNK)], gsem0)
    cp1 = pltpu.async_copy(tok_hbm.at[idx_v.at[1]],
                           rows_v.at[pl.ds(CHUNK, CHUNK)], gsem1)
    pos_cp.wait()
    cp0.wait()
    cp1.wait()

    def add_row(i, carry):
        for j in range(D // 16):
            sl = pl.ds(j * 16, 16)
            rows_v[i, sl] = rows_v[i, sl] + pos_v[i, sl]
        return carry

    lax.fori_loop(0, B_PER_W, add_row, 0)

    pltpu.sync_copy(rows_v, out_hbm.at[pl.ds(base, B_PER_W)])


@jax.jit
def _emb(idx_flat, tok_emb, pos_flat):
    mesh = plsc.VectorSubcoreMesh(core_axis_name="c", subcore_axis_name="s")
    f = functools.partial(
        pl.kernel,
        mesh=mesh,
        out_type=jax.ShapeDtypeStruct((TOKENS, D), jnp.float32),
        scratch_types=[
            pltpu.VMEM((N_CHUNKS, CHUNK), jnp.int32),
            pltpu.VMEM((B_PER_W, D), jnp.float32),
            pltpu.VMEM((B_PER_W, D), jnp.float32),
            pltpu.SemaphoreType.DMA,
            pltpu.SemaphoreType.DMA,
            pltpu.SemaphoreType.DMA,
        ],
    )(_emb_body)
    return f(idx_flat, tok_emb, pos_flat)


def kernel(idx, tok_emb, pos_emb):
    idx_flat = idx.reshape(TOKENS // CHUNK, CHUNK).astype(jnp.int32)
    pos_flat = pos_emb.reshape(-1, D)[:SEQ]
    out = _emb(idx_flat, tok_emb, pos_flat)
    return out.reshape(BATCH, SEQ, D)


# speedup vs baseline: 1.2051x; 1.0234x over previous
import functools

import jax
import jax.numpy as jnp
from jax import lax
from jax.experimental import pallas as pl
from jax.experimental.pallas import tpu as pltpu
from jax.experimental.pallas import tpu_sc as plsc

BATCH = 4
SEQ = 2048
D = 128
TOKENS = BATCH * SEQ
NC, NS = 2, 16
NW = NC * NS
B_PER_W = TOKENS // NW
CHUNK = 64
N_CHUNKS = B_PER_W // CHUNK


def _emb_body(idx_hbm, tok_hbm, pos_hbm, out_hbm, idx_v, rows_v, pos_v,
              psem, wsem, *gsems):
    c = lax.axis_index("c")
    s = lax.axis_index("s")
    wid = s * NC + c
    base = wid * B_PER_W
    t0 = lax.rem(base, SEQ)

    pos_cp = pltpu.async_copy(pos_hbm.at[pl.ds(t0, B_PER_W)], pos_v, psem)

    pltpu.sync_copy(idx_hbm.at[pl.ds(wid * N_CHUNKS, N_CHUNKS)], idx_v)

    gcps = [
        pltpu.async_copy(tok_hbm.at[idx_v.at[k]],
                         rows_v.at[pl.ds(k * CHUNK, CHUNK)], gsems[k])
        for k in range(N_CHUNKS)
    ]
    pos_cp.wait()

    wcps = []
    for k in range(N_CHUNKS):
        gcps[k].wait()

        def add_row(i, carry):
            for j in range(D // 16):
                sl = pl.ds(j * 16, 16)
                rows_v[i, sl] = rows_v[i, sl] + pos_v[i, sl]
            return carry

        lax.fori_loop(k * CHUNK, (k + 1) * CHUNK, add_row, 0)
        wcps.append(
            pltpu.async_copy(rows_v.at[pl.ds(k * CHUNK, CHUNK)],
                             out_hbm.at[pl.ds(base + k * CHUNK, CHUNK)],
                             wsem))
    for cp in wcps:
        cp.wait()


@jax.jit
def _emb(idx_flat, tok_emb, pos_flat):
    mesh = plsc.VectorSubcoreMesh(core_axis_name="c", subcore_axis_name="s")
    f = functools.partial(
        pl.kernel,
        mesh=mesh,
        out_type=jax.ShapeDtypeStruct((TOKENS, D), jnp.float32),
        scratch_types=[
            pltpu.VMEM((N_CHUNKS, CHUNK), jnp.int32),
            pltpu.VMEM((B_PER_W, D), jnp.float32),
            pltpu.VMEM((B_PER_W, D), jnp.float32),
            pltpu.SemaphoreType.DMA,
            pltpu.SemaphoreType.DMA,
        ] + [pltpu.SemaphoreType.DMA] * N_CHUNKS,
    )(_emb_body)
    return f(idx_flat, tok_emb, pos_flat)


def kernel(idx, tok_emb, pos_emb):
    idx_flat = idx.reshape(TOKENS // CHUNK, CHUNK).astype(jnp.int32)
    pos_flat = pos_emb.reshape(-1, D)[:SEQ]
    out = _emb(idx_flat, tok_emb, pos_flat)
    return out.reshape(BATCH, SEQ, D)
